# baseline (speedup 1.0000x reference)
import functools

import jax
import jax.numpy as jnp
from jax import lax
from jax.experimental import pallas as pl
from jax.experimental.pallas import tpu as pltpu
from jax.experimental.pallas import tpu_sc as plsc

_INFO = plsc.get_sparse_core_info()
_NC = _INFO.num_cores
_NS = _INFO.num_subcores
_NW = _NC * _NS
_L = _INFO.num_lanes

_PHASE_COLS = ((0, 1, 2), (0, 2, 1), (1, 2, 0))


def _make_sc_encoder(n, h, w, ncomp):
    pts = n // _NW
    m = 1024
    nchunk = pts // m
    nvec = m // _L
    npair = ncomp // 2
    plane_sz = npair * h * w
    nfeat = 3 * ncomp
    scale_xy = float(w - 1)
    scale_z = float(h - 1)

    mesh = plsc.VectorSubcoreMesh(core_axis_name="c", subcore_axis_name="s")

    @functools.partial(
        pl.kernel,
        out_type=jax.ShapeDtypeStruct((n * nfeat,), jnp.float32),
        mesh=mesh,
        compiler_params=pltpu.CompilerParams(needs_layout_passes=False),
        scratch_types=[
            pltpu.VMEM((3 * plane_sz,), jnp.int32),
            pltpu.VMEM((3 * ncomp * h,), jnp.float32),
            pltpu.VMEM((m,), jnp.float32),
            pltpu.VMEM((m,), jnp.float32),
            pltpu.VMEM((m,), jnp.float32),
            pltpu.VMEM((m * nfeat,), jnp.float32),
        ],
    )
    def encoder(xt_hbm, ptab_hbm, ltab_hbm, out_hbm, tab_v, lt_v, x0_v, x1_v, x2_v, o_v):
        wid = lax.axis_index("s") * _NC + lax.axis_index("c")
        base0 = wid * pts

        pltpu.sync_copy(ptab_hbm, tab_v)
        pltpu.sync_copy(ltab_hbm, lt_v)
        cols = (x0_v, x1_v, x2_v)

        def chunk_body(ch, _):
            gbase = base0 + ch * m
            for c in range(3):
                pltpu.sync_copy(xt_hbm.at[pl.ds(c * n + gbase, m)], cols[c])

            @plsc.parallel_loop(0, nvec, unroll=2)
            def compute(i):
                s = pl.ds(i * _L, _L)
                g0 = x0_v[s]
                g1 = x1_v[s]
                g2 = x2_v[s]
                i0 = g0 * (0.5 * scale_xy) + (0.5 * scale_xy)
                i1 = g1 * (0.5 * scale_xy) + (0.5 * scale_xy)
                i2 = g2 * (0.5 * scale_z) + (0.5 * scale_z)
                c0 = jnp.minimum(i0.astype(jnp.int32), w - 2)
                c1 = jnp.minimum(i1.astype(jnp.int32), h - 2)
                c2 = jnp.minimum(i2.astype(jnp.int32), h - 2)
                f0 = i0 - c0.astype(jnp.float32)
                f1 = i1 - c1.astype(jnp.float32)
                f2 = i2 - c2.astype(jnp.float32)
                ints = (c0, c1, c2)
                fracs = (f0, f1, f2)
                obase = jnp.arange(_L, dtype=jnp.int32) * nfeat + i * (_L * nfeat)

                for p in range(3):
                    ca, cb, cz = _PHASE_COLS[p]
                    xi, yi, zi = ints[ca], ints[cb], ints[cz]
                    fx, fy, fz = fracs[ca], fracs[cb], fracs[cz]
                    f00 = yi * w + xi
                    for pr in range(npair):
                        i00 = f00 + ((p * npair + pr) * h * w)
                        w00 = plsc.load_gather(tab_v, [i00])
                        w01 = plsc.load_gather(tab_v, [i00 + 1])
                        w10 = plsc.load_gather(tab_v, [i00 + w])
                        w11 = plsc.load_gather(tab_v, [i00 + (w + 1)])
                        for half in range(2):
                            c = 2 * pr + half
                            if half == 0:
                                g00 = plsc.bitcast(w00 << 16, jnp.float32)
                                g01 = plsc.bitcast(w01 << 16, jnp.float32)
                                g10 = plsc.bitcast(w10 << 16, jnp.float32)
                                g11 = plsc.bitcast(w11 << 16, jnp.float32)
                            else:
                                msk = jnp.int32(-65536)
                                g00 = plsc.bitcast(w00 & msk, jnp.float32)
                                g01 = plsc.bitcast(w01 & msk, jnp.float32)
                                g10 = plsc.bitcast(w10 & msk, jnp.float32)
                                g11 = plsc.bitcast(w11 & msk, jnp.float32)
                            px0 = g00 + fx * (g01 - g00)
                            px1 = g10 + fx * (g11 - g10)
                            pv = px0 + fy * (px1 - px0)
                            li = zi + ((p * ncomp + c) * h)
                            l0 = plsc.load_gather(lt_v, [li])
                            l1 = plsc.load_gather(lt_v, [li + 1])
                            lv = l0 + fz * (l1 - l0)
                            plsc.store_scatter(
                                o_v, [obase + (p * ncomp + c)], pv * lv
                            )

            pltpu.sync_copy(o_v, out_hbm.at[pl.ds(gbase * nfeat, m * nfeat)])
            return 0

        lax.fori_loop(0, nchunk, chunk_body, 0, unroll=False)

    return encoder


@jax.jit
def kernel(x, plane_coef, line_coef):
    n = x.shape[0]
    nplane, _, h, w = plane_coef.shape
    ncomp = 4
    xt = x.T.reshape(-1)
    pc = plane_coef[:, :ncomp]
    bits = lax.bitcast_convert_type(pc, jnp.uint32)
    rnd = bits + jnp.uint32(0x7FFF) + ((bits >> 16) & jnp.uint32(1))
    pair = rnd.reshape(3, ncomp // 2, 2, h, w)
    word = (pair[:, :, 0] >> 16) | (pair[:, :, 1] & jnp.uint32(0xFFFF0000))
    ptab = lax.bitcast_convert_type(word, jnp.int32).reshape(-1)
    ltab = line_coef[:, :ncomp, :, 0].reshape(-1)
    flat = _make_sc_encoder(n, h, w, ncomp)(xt, ptab, ltab)
    return flat.reshape(n, 3 * ncomp)


# speedup vs baseline: 2.8762x; 2.8762x over previous
import functools

import jax
import jax.numpy as jnp
from jax import lax
from jax.experimental import pallas as pl
from jax.experimental.pallas import tpu as pltpu
from jax.experimental.pallas import tpu_sc as plsc

_INFO = plsc.get_sparse_core_info()
_NC = _INFO.num_cores
_NS = _INFO.num_subcores
_NW = _NC * _NS
_L = _INFO.num_lanes

_PHASE_COLS = ((0, 1, 2), (0, 2, 1), (1, 2, 0))


def _make_sc_encoder(n, h, w, ncomp):
    pts = n // _NW
    m = 1024
    nchunk = pts // m
    nvec = m // _L
    npair = ncomp // 2
    plane_sz = npair * h * w
    nfeat = 3 * ncomp
    scale_xy = float(w - 1)
    scale_z = float(h - 1)

    mesh = plsc.VectorSubcoreMesh(core_axis_name="c", subcore_axis_name="s")

    @functools.partial(
        pl.kernel,
        out_type=jax.ShapeDtypeStruct((n * nfeat,), jnp.float32),
        mesh=mesh,
        compiler_params=pltpu.CompilerParams(needs_layout_passes=False),
        scratch_types=[
            pltpu.VMEM((3 * plane_sz,), jnp.int32),
            pltpu.VMEM((3 * ncomp * h,), jnp.float32),
            pltpu.VMEM((m,), jnp.float32),
            pltpu.VMEM((m,), jnp.float32),
            pltpu.VMEM((m,), jnp.float32),
            pltpu.VMEM((nfeat, m), jnp.float32),
        ],
    )
    def encoder(xt_hbm, ptab_hbm, ltab_hbm, out_hbm, tab_v, lt_v, x0_v, x1_v, x2_v, o_v):
        wid = lax.axis_index("s") * _NC + lax.axis_index("c")
        base0 = wid * pts

        pltpu.sync_copy(ptab_hbm, tab_v)
        pltpu.sync_copy(ltab_hbm, lt_v)
        cols = (x0_v, x1_v, x2_v)

        def chunk_body(ch, _):
            gbase = base0 + ch * m
            for c in range(3):
                pltpu.sync_copy(xt_hbm.at[pl.ds(c * n + gbase, m)], cols[c])

            @plsc.parallel_loop(0, nvec, unroll=2)
            def compute(i):
                s = pl.ds(i * _L, _L)
                g0 = x0_v[s]
                g1 = x1_v[s]
                g2 = x2_v[s]
                i0 = g0 * (0.5 * scale_xy) + (0.5 * scale_xy)
                i1 = g1 * (0.5 * scale_xy) + (0.5 * scale_xy)
                i2 = g2 * (0.5 * scale_z) + (0.5 * scale_z)
                c0 = jnp.minimum(i0.astype(jnp.int32), w - 2)
                c1 = jnp.minimum(i1.astype(jnp.int32), h - 2)
                c2 = jnp.minimum(i2.astype(jnp.int32), h - 2)
                f0 = i0 - c0.astype(jnp.float32)
                f1 = i1 - c1.astype(jnp.float32)
                f2 = i2 - c2.astype(jnp.float32)
                ints = (c0, c1, c2)
                fracs = (f0, f1, f2)
                s_out = pl.ds(i * _L, _L)

                for p in range(3):
                    ca, cb, cz = _PHASE_COLS[p]
                    xi, yi, zi = ints[ca], ints[cb], ints[cz]
                    fx, fy, fz = fracs[ca], fracs[cb], fracs[cz]
                    f00 = yi * w + xi
                    for pr in range(npair):
                        i00 = f00 + ((p * npair + pr) * h * w)
                        w00 = plsc.load_gather(tab_v, [i00])
                        w01 = plsc.load_gather(tab_v, [i00 + 1])
                        w10 = plsc.load_gather(tab_v, [i00 + w])
                        w11 = plsc.load_gather(tab_v, [i00 + (w + 1)])
                        for half in range(2):
                            c = 2 * pr + half
                            if half == 0:
                                g00 = plsc.bitcast(w00 << 16, jnp.float32)
                                g01 = plsc.bitcast(w01 << 16, jnp.float32)
                                g10 = plsc.bitcast(w10 << 16, jnp.float32)
                                g11 = plsc.bitcast(w11 << 16, jnp.float32)
                            else:
                                msk = jnp.int32(-65536)
                                g00 = plsc.bitcast(w00 & msk, jnp.float32)
                                g01 = plsc.bitcast(w01 & msk, jnp.float32)
                                g10 = plsc.bitcast(w10 & msk, jnp.float32)
                                g11 = plsc.bitcast(w11 & msk, jnp.float32)
                            px0 = g00 + fx * (g01 - g00)
                            px1 = g10 + fx * (g11 - g10)
                            pv = px0 + fy * (px1 - px0)
                            li = zi + ((p * ncomp + c) * h)
                            l0 = plsc.load_gather(lt_v, [li])
                            l1 = plsc.load_gather(lt_v, [li + 1])
                            lv = l0 + fz * (l1 - l0)
                            o_v[p * ncomp + c, s_out] = pv * lv

            for j in range(nfeat):
                pltpu.sync_copy(o_v.at[j], out_hbm.at[pl.ds(j * n + gbase, m)])
            return 0

        lax.fori_loop(0, nchunk, chunk_body, 0, unroll=False)

    return encoder


@jax.jit
def kernel(x, plane_coef, line_coef):
    n = x.shape[0]
    nplane, _, h, w = plane_coef.shape
    ncomp = 4
    xt = x.T.reshape(-1)
    pc = plane_coef[:, :ncomp]
    bits = lax.bitcast_convert_type(pc, jnp.uint32)
    rnd = bits + jnp.uint32(0x7FFF) + ((bits >> 16) & jnp.uint32(1))
    pair = rnd.reshape(3, ncomp // 2, 2, h, w)
    word = (pair[:, :, 0] >> 16) | (pair[:, :, 1] & jnp.uint32(0xFFFF0000))
    ptab = lax.bitcast_convert_type(word, jnp.int32).reshape(-1)
    ltab = line_coef[:, :ncomp, :, 0].reshape(-1)
    flat = _make_sc_encoder(n, h, w, ncomp)(xt, ptab, ltab)
    return flat.reshape(3 * ncomp, n).T
